# initial kernel scaffold (unmeasured)
import jax
import jax.numpy as jnp
from jax import lax
from jax.experimental import pallas as pl
from jax.experimental.pallas import tpu as pltpu


def kernel(ids, E):
    T = ids.shape[0]
    V, D = E.shape

    x = lax.axis_index("x")
    y = lax.axis_index("y")
    z = lax.axis_index("z")

    local = ids - x * V
    mask = (local >= 0) & (local < V)
    safe = jnp.where(mask, local, 0)
    part = jnp.where(mask[:, None], E[safe, :], 0.0).astype(jnp.bfloat16)

    def body(part_ref, out_ref, recv_ref, send_sem, recv_sem):
        barrier = pltpu.get_barrier_semaphore()
        pl.semaphore_signal(
            barrier, inc=1,
            device_id=(1 - x, y, z), device_id_type=pl.DeviceIdType.MESH,
        )
        pl.semaphore_wait(barrier, 1)

        rdma = pltpu.make_async_remote_copy(
            src_ref=part_ref,
            dst_ref=recv_ref,
            send_sem=send_sem,
            recv_sem=recv_sem,
            device_id=(1 - x, y, z),
            device_id_type=pl.DeviceIdType.MESH,
        )
        rdma.start()
        rdma.wait()
        out_ref[...] = part_ref[...] + recv_ref[...]

    out = pl.pallas_call(
        body,
        out_shape=jax.ShapeDtypeStruct((T, D), jnp.bfloat16),
        in_specs=[pl.BlockSpec(memory_space=pltpu.VMEM)],
        out_specs=pl.BlockSpec(memory_space=pltpu.VMEM),
        scratch_shapes=[
            pltpu.VMEM((T, D), jnp.bfloat16),
            pltpu.SemaphoreType.DMA,
            pltpu.SemaphoreType.DMA,
        ],
        compiler_params=pltpu.CompilerParams(collective_id=0),
    )(part)
    return out.astype(jnp.float32)


# baseline (device time: 363314 ns/iter reference)
import jax
import jax.numpy as jnp
from jax import lax
from jax.experimental import pallas as pl
from jax.experimental.pallas import tpu as pltpu


def kernel(ids, E):
    T = ids.shape[0]
    V, D = E.shape

    x = lax.axis_index("x")
    y = lax.axis_index("y")
    z = lax.axis_index("z")

    local = ids - x * V
    mask = (local >= 0) & (local < V)
    safe = jnp.where(mask, local, 0)
    part = jnp.where(mask[:, None], E[safe, :], 0.0).astype(jnp.bfloat16)

    def body(part_ref, out_ref, recv_ref, send_sem, recv_sem):
        x = lax.axis_index("x")
        y = lax.axis_index("y")
        z = lax.axis_index("z")
        barrier = pltpu.get_barrier_semaphore()
        pl.semaphore_signal(
            barrier, inc=1,
            device_id=(1 - x, y, z), device_id_type=pl.DeviceIdType.MESH,
        )
        pl.semaphore_wait(barrier, 1)

        rdma = pltpu.make_async_remote_copy(
            src_ref=part_ref,
            dst_ref=recv_ref,
            send_sem=send_sem,
            recv_sem=recv_sem,
            device_id=(1 - x, y, z),
            device_id_type=pl.DeviceIdType.MESH,
        )
        rdma.start()
        rdma.wait()
        out_ref[...] = part_ref[...] + recv_ref[...]

    out = pl.pallas_call(
        body,
        out_shape=jax.ShapeDtypeStruct((T, D), jnp.bfloat16),
        in_specs=[pl.BlockSpec(memory_space=pltpu.VMEM)],
        out_specs=pl.BlockSpec(memory_space=pltpu.VMEM),
        scratch_shapes=[
            pltpu.VMEM((T, D), jnp.bfloat16),
            pltpu.SemaphoreType.DMA,
            pltpu.SemaphoreType.DMA,
        ],
        compiler_params=pltpu.CompilerParams(collective_id=0),
    )(part)
    return out.astype(jnp.float32)


# device time: 187621 ns/iter; 1.9364x vs baseline; 1.9364x over previous
import jax
import jax.numpy as jnp
from jax import lax
from jax.experimental import pallas as pl
from jax.experimental.pallas import tpu as pltpu

K = 4


def kernel(ids, E):
    T = ids.shape[0]
    V, D = E.shape
    Q = T // 4
    CH = Q // K

    x = lax.axis_index("x")
    y = lax.axis_index("y")
    z = lax.axis_index("z")
    q = 2 * y + z

    ids_q = lax.dynamic_slice(ids, (q * Q,), (Q,))
    local = ids_q - x * V
    mask = (local >= 0) & (local < V)
    safe = jnp.where(mask, local, 0)
    part = jnp.where(mask[:, None], E[safe, :], 0.0).astype(jnp.bfloat16)

    def body(part_ref, out_ref, recv_a, half, recv_c,
             a_send, a_recv, b_send, b_recv, c_send, c_recv):
        x = lax.axis_index("x")
        y = lax.axis_index("y")
        z = lax.axis_index("z")
        q = 2 * y + z
        q_y = 2 * (1 - y) + z
        q_z = 2 * y + (1 - z)
        q_yz = 2 * (1 - y) + (1 - z)
        x_peer = (1 - x, y, z)
        y_peer = (x, 1 - y, z)
        z_peer = (x, y, 1 - z)

        barrier = pltpu.get_barrier_semaphore()
        for peer in (x_peer, y_peer, z_peer):
            pl.semaphore_signal(
                barrier, inc=1,
                device_id=peer, device_id_type=pl.DeviceIdType.MESH,
            )
        pl.semaphore_wait(barrier, 3)

        def ch(k):
            return pl.ds(k * CH, CH)

        a_rdmas = []
        for k in range(K):
            r = pltpu.make_async_remote_copy(
                src_ref=part_ref.at[ch(k)],
                dst_ref=recv_a.at[ch(k)],
                send_sem=a_send.at[k], recv_sem=a_recv.at[k],
                device_id=x_peer, device_id_type=pl.DeviceIdType.MESH,
            )
            r.start()
            a_rdmas.append(r)

        b_rdmas, c1_rdmas = [], []
        for k in range(K):
            a_rdmas[k].wait()
            red = part_ref[ch(k), :] + recv_a[ch(k), :]
            half[0, ch(k), :] = red
            out_ref[pl.ds(q * Q + k * CH, CH), :] = red.astype(jnp.float32)
            rb = pltpu.make_async_remote_copy(
                src_ref=half.at[0, ch(k)],
                dst_ref=half.at[1, ch(k)],
                send_sem=b_send.at[k], recv_sem=b_recv.at[k],
                device_id=y_peer, device_id_type=pl.DeviceIdType.MESH,
            )
            rb.start()
            b_rdmas.append(rb)
            rc = pltpu.make_async_remote_copy(
                src_ref=half.at[0, ch(k)],
                dst_ref=recv_c.at[0, ch(k)],
                send_sem=c_send.at[0, k], recv_sem=c_recv.at[0, k],
                device_id=z_peer, device_id_type=pl.DeviceIdType.MESH,
            )
            rc.start()
            c1_rdmas.append(rc)

        c2_rdmas = []
        for k in range(K):
            b_rdmas[k].wait()
            out_ref[pl.ds(q_y * Q + k * CH, CH), :] = (
                half[1, ch(k), :].astype(jnp.float32))
            rc = pltpu.make_async_remote_copy(
                src_ref=half.at[1, ch(k)],
                dst_ref=recv_c.at[1, ch(k)],
                send_sem=c_send.at[1, k], recv_sem=c_recv.at[1, k],
                device_id=z_peer, device_id_type=pl.DeviceIdType.MESH,
            )
            rc.start()
            c2_rdmas.append(rc)

        for k in range(K):
            c1_rdmas[k].wait()
            out_ref[pl.ds(q_z * Q + k * CH, CH), :] = (
                recv_c[0, ch(k), :].astype(jnp.float32))
        for k in range(K):
            c2_rdmas[k].wait()
            out_ref[pl.ds(q_yz * Q + k * CH, CH), :] = (
                recv_c[1, ch(k), :].astype(jnp.float32))

    out = pl.pallas_call(
        body,
        out_shape=jax.ShapeDtypeStruct((T, D), jnp.float32),
        in_specs=[pl.BlockSpec(memory_space=pltpu.VMEM)],
        out_specs=pl.BlockSpec(memory_space=pltpu.VMEM),
        scratch_shapes=[
            pltpu.VMEM((Q, D), jnp.bfloat16),
            pltpu.VMEM((2, Q, D), jnp.bfloat16),
            pltpu.VMEM((2, Q, D), jnp.bfloat16),
            pltpu.SemaphoreType.DMA((K,)),
            pltpu.SemaphoreType.DMA((K,)),
            pltpu.SemaphoreType.DMA((K,)),
            pltpu.SemaphoreType.DMA((K,)),
            pltpu.SemaphoreType.DMA((2, K)),
            pltpu.SemaphoreType.DMA((2, K)),
        ],
        compiler_params=pltpu.CompilerParams(
            collective_id=0, vmem_limit_bytes=100 * 1024 * 1024),
    )(part)
    return out


# device time: 166845 ns/iter; 2.1776x vs baseline; 1.1245x over previous
import jax
import jax.numpy as jnp
from jax import lax
from jax.experimental import pallas as pl
from jax.experimental.pallas import tpu as pltpu

K = 8


def kernel(ids, E):
    T = ids.shape[0]
    V, D = E.shape
    Q = T // 4
    CH = Q // K

    x = lax.axis_index("x")
    y = lax.axis_index("y")
    z = lax.axis_index("z")
    q = 2 * y + z

    ids_q = lax.dynamic_slice(ids, (q * Q,), (Q,))
    local = ids_q - x * V
    mask = (local >= 0) & (local < V)
    safe = jnp.where(mask, local, 0)
    part = jnp.where(mask[:, None], E[safe, :], 0.0).astype(jnp.bfloat16)

    def body(part_ref, out_ref, recv_a, half, recv_c,
             a_send, a_recv, b_send, b_recv, c_send, c_recv):
        x = lax.axis_index("x")
        y = lax.axis_index("y")
        z = lax.axis_index("z")
        q = 2 * y + z
        q_y = 2 * (1 - y) + z
        q_z = 2 * y + (1 - z)
        q_yz = 2 * (1 - y) + (1 - z)
        x_peer = (1 - x, y, z)
        y_peer = (x, 1 - y, z)
        z_peer = (x, y, 1 - z)

        barrier = pltpu.get_barrier_semaphore()
        for peer in (x_peer, y_peer, z_peer):
            pl.semaphore_signal(
                barrier, inc=1,
                device_id=peer, device_id_type=pl.DeviceIdType.MESH,
            )
        pl.semaphore_wait(barrier, 3)

        def ch(k):
            return pl.ds(k * CH, CH)

        a_rdmas = []
        for k in range(K):
            r = pltpu.make_async_remote_copy(
                src_ref=part_ref.at[ch(k)],
                dst_ref=recv_a.at[ch(k)],
                send_sem=a_send.at[k], recv_sem=a_recv.at[k],
                device_id=x_peer, device_id_type=pl.DeviceIdType.MESH,
            )
            r.start()
            a_rdmas.append(r)

        b_rdmas, c1_rdmas = [], []
        for k in range(K):
            a_rdmas[k].wait()
            red = part_ref[ch(k), :] + recv_a[ch(k), :]
            half[0, ch(k), :] = red
            out_ref[pl.ds(q * Q + k * CH, CH), :] = red
            rb = pltpu.make_async_remote_copy(
                src_ref=half.at[0, ch(k)],
                dst_ref=half.at[1, ch(k)],
                send_sem=b_send.at[k], recv_sem=b_recv.at[k],
                device_id=y_peer, device_id_type=pl.DeviceIdType.MESH,
            )
            rb.start()
            b_rdmas.append(rb)
            rc = pltpu.make_async_remote_copy(
                src_ref=half.at[0, ch(k)],
                dst_ref=recv_c.at[0, ch(k)],
                send_sem=c_send.at[0, k], recv_sem=c_recv.at[0, k],
                device_id=z_peer, device_id_type=pl.DeviceIdType.MESH,
            )
            rc.start()
            c1_rdmas.append(rc)

        c2_rdmas = []
        for k in range(K):
            b_rdmas[k].wait()
            out_ref[pl.ds(q_y * Q + k * CH, CH), :] = half[1, ch(k), :]
            rc = pltpu.make_async_remote_copy(
                src_ref=half.at[1, ch(k)],
                dst_ref=recv_c.at[1, ch(k)],
                send_sem=c_send.at[1, k], recv_sem=c_recv.at[1, k],
                device_id=z_peer, device_id_type=pl.DeviceIdType.MESH,
            )
            rc.start()
            c2_rdmas.append(rc)

        for k in range(K):
            c1_rdmas[k].wait()
            out_ref[pl.ds(q_z * Q + k * CH, CH), :] = recv_c[0, ch(k), :]
        for k in range(K):
            c2_rdmas[k].wait()
            out_ref[pl.ds(q_yz * Q + k * CH, CH), :] = recv_c[1, ch(k), :]

    out = pl.pallas_call(
        body,
        out_shape=jax.ShapeDtypeStruct((T, D), jnp.bfloat16),
        in_specs=[pl.BlockSpec(memory_space=pltpu.VMEM)],
        out_specs=pl.BlockSpec(memory_space=pltpu.VMEM),
        scratch_shapes=[
            pltpu.VMEM((Q, D), jnp.bfloat16),
            pltpu.VMEM((2, Q, D), jnp.bfloat16),
            pltpu.VMEM((2, Q, D), jnp.bfloat16),
            pltpu.SemaphoreType.DMA((K,)),
            pltpu.SemaphoreType.DMA((K,)),
            pltpu.SemaphoreType.DMA((K,)),
            pltpu.SemaphoreType.DMA((K,)),
            pltpu.SemaphoreType.DMA((2, K)),
            pltpu.SemaphoreType.DMA((2, K)),
        ],
        compiler_params=pltpu.CompilerParams(
            collective_id=0, vmem_limit_bytes=100 * 1024 * 1024),
    )(part)
    return out


# device time: 157456 ns/iter; 2.3074x vs baseline; 1.0596x over previous
import jax
import jax.numpy as jnp
from jax import lax
from jax.experimental import pallas as pl
from jax.experimental.pallas import tpu as pltpu

K = 8


def kernel(ids, E):
    T = ids.shape[0]
    V, D = E.shape
    Q = T // 4
    CH = Q // K

    x = lax.axis_index("x")
    y = lax.axis_index("y")
    z = lax.axis_index("z")
    q = 2 * y + z

    ids_q = lax.dynamic_slice(ids, (q * Q,), (Q,))
    local = ids_q - x * V
    lids = jnp.clip(local, 0, V - 1).astype(jnp.int32)
    maskq = ((local >= 0) & (local < V)).astype(jnp.bfloat16).reshape(Q, 1)

    def body(lids_ref, mask_ref, e_ref, out_ref,
             part32, part, recv_a, half, recv_c,
             g_sem, a_send, a_recv, b_send, b_recv, c_send, c_recv):
        x = lax.axis_index("x")
        y = lax.axis_index("y")
        z = lax.axis_index("z")
        q = 2 * y + z
        q_y = 2 * (1 - y) + z
        q_z = 2 * y + (1 - z)
        q_yz = 2 * (1 - y) + (1 - z)
        x_peer = (1 - x, y, z)
        y_peer = (x, 1 - y, z)
        z_peer = (x, y, 1 - z)

        def ch(k):
            return pl.ds(k * CH, CH)

        def issue_gather(k):
            def one(i, carry):
                tok = k * CH + i
                idx = lids_ref[tok]
                pltpu.make_async_copy(
                    e_ref.at[pl.ds(idx, 1), :],
                    part32.at[pl.ds(tok, 1), :],
                    g_sem.at[k],
                ).start()
                return carry
            lax.fori_loop(0, CH, one, 0)

        def wait_gather(k):
            def one(i, carry):
                pltpu.make_async_copy(
                    e_ref.at[pl.ds(0, 1), :],
                    part32.at[pl.ds(0, 1), :],
                    g_sem.at[k],
                ).wait()
                return carry
            lax.fori_loop(0, CH, one, 0)

        issue_gather(0)

        barrier = pltpu.get_barrier_semaphore()
        for peer in (x_peer, y_peer, z_peer):
            pl.semaphore_signal(
                barrier, inc=1,
                device_id=peer, device_id_type=pl.DeviceIdType.MESH,
            )
        pl.semaphore_wait(barrier, 3)

        a_rdmas = []
        for k in range(K):
            if k + 1 < K:
                issue_gather(k + 1)
            wait_gather(k)
            part[ch(k), :] = part32[ch(k), :].astype(jnp.bfloat16)
            r = pltpu.make_async_remote_copy(
                src_ref=part.at[ch(k)],
                dst_ref=recv_a.at[ch(k)],
                send_sem=a_send.at[k], recv_sem=a_recv.at[k],
                device_id=x_peer, device_id_type=pl.DeviceIdType.MESH,
            )
            r.start()
            a_rdmas.append(r)

        b_rdmas, c1_rdmas = [], []
        for k in range(K):
            a_rdmas[k].wait()
            red = jnp.where(
                mask_ref[ch(k), :] != 0, part[ch(k), :], recv_a[ch(k), :])
            half[0, ch(k), :] = red
            out_ref[pl.ds(q * Q + k * CH, CH), :] = red
            rb = pltpu.make_async_remote_copy(
                src_ref=half.at[0, ch(k)],
                dst_ref=half.at[1, ch(k)],
                send_sem=b_send.at[k], recv_sem=b_recv.at[k],
                device_id=y_peer, device_id_type=pl.DeviceIdType.MESH,
            )
            rb.start()
            b_rdmas.append(rb)
            rc = pltpu.make_async_remote_copy(
                src_ref=half.at[0, ch(k)],
                dst_ref=recv_c.at[0, ch(k)],
                send_sem=c_send.at[0, k], recv_sem=c_recv.at[0, k],
                device_id=z_peer, device_id_type=pl.DeviceIdType.MESH,
            )
            rc.start()
            c1_rdmas.append(rc)

        c2_rdmas = []
        for k in range(K):
            b_rdmas[k].wait()
            out_ref[pl.ds(q_y * Q + k * CH, CH), :] = half[1, ch(k), :]
            rc = pltpu.make_async_remote_copy(
                src_ref=half.at[1, ch(k)],
                dst_ref=recv_c.at[1, ch(k)],
                send_sem=c_send.at[1, k], recv_sem=c_recv.at[1, k],
                device_id=z_peer, device_id_type=pl.DeviceIdType.MESH,
            )
            rc.start()
            c2_rdmas.append(rc)

        for k in range(K):
            c1_rdmas[k].wait()
            out_ref[pl.ds(q_z * Q + k * CH, CH), :] = recv_c[0, ch(k), :]
        for k in range(K):
            c2_rdmas[k].wait()
            out_ref[pl.ds(q_yz * Q + k * CH, CH), :] = recv_c[1, ch(k), :]

    out = pl.pallas_call(
        body,
        out_shape=jax.ShapeDtypeStruct((T, D), jnp.bfloat16),
        in_specs=[
            pl.BlockSpec(memory_space=pltpu.SMEM),
            pl.BlockSpec(memory_space=pltpu.VMEM),
            pl.BlockSpec(memory_space=pl.ANY),
        ],
        out_specs=pl.BlockSpec(memory_space=pltpu.VMEM),
        scratch_shapes=[
            pltpu.VMEM((Q, D), jnp.float32),
            pltpu.VMEM((Q, D), jnp.bfloat16),
            pltpu.VMEM((Q, D), jnp.bfloat16),
            pltpu.VMEM((2, Q, D), jnp.bfloat16),
            pltpu.VMEM((2, Q, D), jnp.bfloat16),
            pltpu.SemaphoreType.DMA((K,)),
            pltpu.SemaphoreType.DMA((K,)),
            pltpu.SemaphoreType.DMA((K,)),
            pltpu.SemaphoreType.DMA((K,)),
            pltpu.SemaphoreType.DMA((K,)),
            pltpu.SemaphoreType.DMA((2, K)),
            pltpu.SemaphoreType.DMA((2, K)),
        ],
        compiler_params=pltpu.CompilerParams(
            collective_id=0, vmem_limit_bytes=100 * 1024 * 1024),
    )(lids, maskq, E)
    return out


# device time: 122377 ns/iter; 2.9688x vs baseline; 1.2866x over previous
import jax
import jax.numpy as jnp
from jax import lax
from jax.experimental import pallas as pl
from jax.experimental.pallas import tpu as pltpu

K = 8


def kernel(ids, E):
    T = ids.shape[0]
    V, D = E.shape
    Q = T // 4
    CH = Q // K
    H = CH // 2

    x = lax.axis_index("x")
    y = lax.axis_index("y")
    z = lax.axis_index("z")
    q = 2 * y + z

    ids_q = lax.dynamic_slice(ids, (q * Q,), (Q,))
    local = ids_q - x * V
    own = (local >= 0) & (local < V)
    lids = jnp.clip(local, 0, V - 1).astype(jnp.int32)
    owni = own.astype(jnp.int32)
    cnts = owni.reshape(K, CH).sum(axis=1)
    maskq = own.astype(jnp.bfloat16).reshape(Q, 1)

    def body(lids_ref, own_ref, cnt_ref, mask_ref, e_ref, out_ref,
             part32, part, recv_a, redbuf,
             g_sem, r_sem, a_send, a_recv, b_send, b_recv,
             c1_send, c1_recv, c2_send, c2_recv, d_send, d_recv):
        x = lax.axis_index("x")
        y = lax.axis_index("y")
        z = lax.axis_index("z")
        q = 2 * y + z
        q_y = 2 * (1 - y) + z
        q_z = 2 * y + (1 - z)
        x_peer = (1 - x, y, z)
        y_peer = (x, 1 - y, z)
        z_peer = (x, y, 1 - z)

        def ch(k):
            return pl.ds(k * CH, CH)

        def issue_gather(k):
            def one(i, carry):
                tok = k * CH + i

                @pl.when(own_ref[tok] == 1)
                def _():
                    idx = lids_ref[tok]
                    pltpu.make_async_copy(
                        e_ref.at[pl.ds(idx, 1), :],
                        part32.at[pl.ds(tok, 1), :],
                        g_sem.at[k],
                    ).start()
                return carry
            lax.fori_loop(0, CH, one, 0)

        def wait_gather(k):
            def one(i, carry):
                pltpu.make_async_copy(
                    e_ref.at[pl.ds(0, 1), :],
                    part32.at[pl.ds(0, 1), :],
                    g_sem.at[k],
                ).wait()
                return carry
            lax.fori_loop(0, cnt_ref[k], one, 0)

        issue_gather(0)
        issue_gather(1)

        barrier = pltpu.get_barrier_semaphore()
        for peer in (x_peer, y_peer, z_peer):
            pl.semaphore_signal(
                barrier, inc=1,
                device_id=peer, device_id_type=pl.DeviceIdType.MESH,
            )
        pl.semaphore_wait(barrier, 3)

        a_rdmas = []
        for k in range(K):
            if k + 2 < K:
                issue_gather(k + 2)
            wait_gather(k)
            part[ch(k), :] = part32[ch(k), :].astype(jnp.bfloat16)
            r = pltpu.make_async_remote_copy(
                src_ref=part.at[ch(k)],
                dst_ref=recv_a.at[ch(k)],
                send_sem=a_send.at[k], recv_sem=a_recv.at[k],
                device_id=x_peer, device_id_type=pl.DeviceIdType.MESH,
            )
            r.start()
            a_rdmas.append(r)

        b_rdmas, c1_rdmas, c2_rdmas, d_rdmas = [], [], [], []

        def on_b(j):
            b_rdmas[j].wait()
            sl = pl.ds(q_y * Q + j * CH, H)
            rc = pltpu.make_async_remote_copy(
                src_ref=out_ref.at[sl],
                dst_ref=out_ref.at[sl],
                send_sem=c2_send.at[j], recv_sem=c2_recv.at[j],
                device_id=z_peer, device_id_type=pl.DeviceIdType.MESH,
            )
            rc.start()
            c2_rdmas.append(rc)

        def on_c1(j):
            c1_rdmas[j].wait()
            sl = pl.ds(q_z * Q + j * CH + H, H)
            rd = pltpu.make_async_remote_copy(
                src_ref=out_ref.at[sl],
                dst_ref=out_ref.at[sl],
                send_sem=d_send.at[j], recv_sem=d_recv.at[j],
                device_id=y_peer, device_id_type=pl.DeviceIdType.MESH,
            )
            rd.start()
            d_rdmas.append(rd)

        for k in range(K):
            a_rdmas[k].wait()
            red = jnp.where(
                mask_ref[ch(k), :] != 0, part[ch(k), :], recv_a[ch(k), :])
            sl = pl.ds(q * Q + k * CH, CH)
            redbuf[ch(k), :] = red
            pltpu.make_async_copy(
                redbuf.at[ch(k)], out_ref.at[sl], r_sem.at[k]).start()
            rb = pltpu.make_async_remote_copy(
                src_ref=redbuf.at[ch(k)],
                dst_ref=out_ref.at[sl],
                send_sem=b_send.at[k], recv_sem=b_recv.at[k],
                device_id=y_peer, device_id_type=pl.DeviceIdType.MESH,
            )
            rb.start()
            b_rdmas.append(rb)
            rc = pltpu.make_async_remote_copy(
                src_ref=redbuf.at[ch(k)],
                dst_ref=out_ref.at[sl],
                send_sem=c1_send.at[k], recv_sem=c1_recv.at[k],
                device_id=z_peer, device_id_type=pl.DeviceIdType.MESH,
            )
            rc.start()
            c1_rdmas.append(rc)
            if k >= 1:
                on_b(k - 1)
                on_c1(k - 1)

        for j in range(K - 1, K):
            on_b(j)
            on_c1(j)

        for k in range(K):
            c2_rdmas[k].wait()
            d_rdmas[k].wait()
            pltpu.make_async_copy(
                redbuf.at[ch(k)],
                out_ref.at[pl.ds(q * Q + k * CH, CH)],
                r_sem.at[k],
            ).wait()

    out = pl.pallas_call(
        body,
        out_shape=jax.ShapeDtypeStruct((T, D), jnp.bfloat16),
        in_specs=[
            pl.BlockSpec(memory_space=pltpu.SMEM),
            pl.BlockSpec(memory_space=pltpu.SMEM),
            pl.BlockSpec(memory_space=pltpu.SMEM),
            pl.BlockSpec(memory_space=pltpu.VMEM),
            pl.BlockSpec(memory_space=pl.ANY),
        ],
        out_specs=pl.BlockSpec(memory_space=pl.ANY),
        scratch_shapes=[
            pltpu.VMEM((Q, D), jnp.float32),
            pltpu.VMEM((Q, D), jnp.bfloat16),
            pltpu.VMEM((Q, D), jnp.bfloat16),
            pltpu.VMEM((Q, D), jnp.bfloat16),
            pltpu.SemaphoreType.DMA((K,)),
            pltpu.SemaphoreType.DMA((K,)),
            pltpu.SemaphoreType.DMA((K,)),
            pltpu.SemaphoreType.DMA((K,)),
            pltpu.SemaphoreType.DMA((K,)),
            pltpu.SemaphoreType.DMA((K,)),
            pltpu.SemaphoreType.DMA((K,)),
            pltpu.SemaphoreType.DMA((K,)),
            pltpu.SemaphoreType.DMA((K,)),
            pltpu.SemaphoreType.DMA((K,)),
            pltpu.SemaphoreType.DMA((K,)),
            pltpu.SemaphoreType.DMA((K,)),
        ],
        compiler_params=pltpu.CompilerParams(
            collective_id=0, vmem_limit_bytes=100 * 1024 * 1024),
    )(lids, owni, cnts, maskq, E)
    return out
